# baseline (speedup 1.0000x reference)
import functools

import jax
import jax.numpy as jnp
from jax import lax
from jax.experimental import pallas as pl
from jax.experimental.pallas import tpu as pltpu
from jax.experimental.pallas import tpu_sc as plsc

N_NODES = 10000
N_EDGES = 320000
D_FEAT = 128
D_HID = 16
N_CLASSES = 7

NC = 2
NS = 16
NW = NC * NS
NPAD = 10240
PK = 8
PROWS = NPAD // PK
XROWS = N_NODES // PK
EPW = N_EDGES // NW
CHUNK = 125
NCHUNK = EPW // CHUNK
NB = 4
ROWS_PT = NPAD // NS
PROWS_PT = PROWS // NS


def _mesh():
    return plsc.VectorSubcoreMesh(core_axis_name="c", subcore_axis_name="s")


_SC_PARAMS = pltpu.CompilerParams(use_tc_tiling_on_sc=False,
                                  needs_layout_passes=False)


@functools.partial(
    pl.kernel,
    out_type=jax.ShapeDtypeStruct((NC, PROWS, 128), jnp.float32),
    mesh=_mesh(),
    compiler_params=_SC_PARAMS,
    scratch_types=[
        pltpu.VMEM((NCHUNK, CHUNK), jnp.int32),
        pltpu.VMEM((CHUNK,), jnp.float32),
        pltpu.VMEM((ROWS_PT,), jnp.float32),
        pltpu.VMEM((PROWS_PT, 128), jnp.float32),
        pltpu.VMEM_SHARED((NPAD,), jnp.float32),
        pltpu.SemaphoreType.DMA,
    ],
)
def _sc_degree(ei_hbm, out_hbm, dstb, onesb, degt, repb, degsh, sem):
    c = lax.axis_index("c")
    s = lax.axis_index("s")
    wid = c * NS + s
    row0 = s * ROWS_PT

    @pl.loop(0, ROWS_PT // 16)
    def _(i):
        degt[pl.ds(i * 16, 16)] = jnp.zeros((16,), jnp.float32)

    for o in (0, 16, 32, 48, 64, 80, 96, CHUNK - 16):
        onesb[pl.ds(o, 16)] = jnp.ones((16,), jnp.float32)

    pltpu.sync_copy(degt, degsh.at[pl.ds(row0, ROWS_PT)])
    pltpu.sync_copy(ei_hbm.at[1, wid], dstb)
    plsc.subcore_barrier()

    @pl.loop(0, NCHUNK, step=8)
    def _(j0):
        @pl.loop(0, 8)
        def _(b):
            pltpu.async_copy(onesb, degsh.at[dstb.at[j0 + b]], sem, add=True)

        @pl.loop(0, 8)
        def _(b):
            pltpu.make_async_copy(onesb, degsh.at[dstb.at[0]], sem).wait()

    plsc.subcore_barrier()

    pltpu.sync_copy(degsh.at[pl.ds(row0, ROWS_PT)], degt)

    @pl.loop(0, ROWS_PT // 16)
    def _(ii):
        d16 = degt[pl.ds(ii * 16, 16)]
        for k in range(16):
            repb[ii * 2 + k // PK, pl.ds((k % PK) * 16, 16)] = (
                jnp.zeros((16,), jnp.float32) + d16[k])

    pltpu.sync_copy(repb, out_hbm.at[c, pl.ds(s * PROWS_PT, PROWS_PT)])


@functools.partial(
    pl.kernel,
    out_type=jax.ShapeDtypeStruct((NC, PROWS, 128), jnp.float32),
    mesh=_mesh(),
    compiler_params=_SC_PARAMS,
    scratch_types=[
        pltpu.VMEM((NCHUNK, CHUNK), jnp.int32),
        pltpu.VMEM((NCHUNK, CHUNK), jnp.int32),
        [pltpu.VMEM((CHUNK, D_HID), jnp.float32) for _ in range(NB)],
        pltpu.VMEM((PROWS_PT, 128), jnp.float32),
        pltpu.VMEM((ROWS_PT, D_HID), jnp.float32),
        pltpu.VMEM((ROWS_PT, D_HID), jnp.float32),
        pltpu.VMEM_SHARED((NPAD, D_HID), jnp.float32),
        pltpu.VMEM_SHARED((NPAD, D_HID), jnp.float32),
        pltpu.VMEM_SHARED((NPAD, D_HID), jnp.float32),
        [pltpu.SemaphoreType.DMA for _ in range(NB)],
        [pltpu.SemaphoreType.DMA for _ in range(NB)],
    ],
)
def _sc_agg(g_hbm, ei_hbm, out_hbm, srcb, dstb, rowsb, pbuf, ubuf, ubuf2,
            aggsh, aggsh2, gsh, gsem, ssem):
    c = lax.axis_index("c")
    s = lax.axis_index("s")
    wid = c * NS + s
    row0 = s * ROWS_PT
    prow0 = s * PROWS_PT

    pltpu.sync_copy(g_hbm.at[pl.ds(prow0, PROWS_PT)], pbuf)
    pltpu.sync_copy(ei_hbm.at[0, wid], srcb)
    pltpu.sync_copy(ei_hbm.at[1, wid], dstb)

    @pl.loop(0, PROWS_PT)
    def _(r):
        for k in range(PK):
            ubuf[r * PK + k, :] = pbuf[r, pl.ds(k * 16, 16)]

    pltpu.sync_copy(ubuf, gsh.at[pl.ds(row0, ROWS_PT)])

    @pl.loop(0, ROWS_PT)
    def _(i):
        ubuf[i, :] = jnp.zeros((D_HID,), jnp.float32)

    pltpu.sync_copy(ubuf, aggsh.at[pl.ds(row0, ROWS_PT)])
    pltpu.sync_copy(ubuf, aggsh2.at[pl.ds(row0, ROWS_PT)])
    plsc.subcore_barrier()

    def start_gather(j, b):
        pltpu.async_copy(gsh.at[srcb.at[j]], rowsb[b], gsem[b])

    def wait_gather(b):
        pltpu.make_async_copy(gsh.at[srcb.at[0]], rowsb[b], gsem[b]).wait()

    def start_scatter(j, b):
        bank = aggsh if b % 2 == 0 else aggsh2
        pltpu.async_copy(rowsb[b], bank.at[dstb.at[j]], ssem[b], add=True)

    def wait_scatter(b):
        bank = aggsh if b % 2 == 0 else aggsh2
        pltpu.make_async_copy(rowsb[b], bank.at[dstb.at[0]], ssem[b]).wait()

    for b in range(NB):
        start_gather(b, b)

    @pl.loop(0, (NCHUNK - NB) // NB)
    def _(gg):
        base = gg * NB
        for b in range(NB):
            j = base + b
            wait_gather(b)
            start_scatter(j, b)
            wait_scatter(b)
            start_gather(j + NB, b)

    for b in range(NB):
        j = NCHUNK - NB + b
        wait_gather(b)
        start_scatter(j, b)
        wait_scatter(b)

    plsc.subcore_barrier()

    pltpu.sync_copy(aggsh.at[pl.ds(row0, ROWS_PT)], ubuf)
    pltpu.sync_copy(aggsh2.at[pl.ds(row0, ROWS_PT)], ubuf2)

    @pl.loop(0, PROWS_PT)
    def _(r):
        for k in range(PK):
            pbuf[r, pl.ds(k * 16, 16)] = (ubuf[r * PK + k, :]
                                          + ubuf2[r * PK + k, :])

    pltpu.sync_copy(pbuf, out_hbm.at[c, pl.ds(prow0, PROWS_PT)])


def _tc_matmul_body(x_ref, w_ref, o_ref):
    for k in range(PK):
        o_ref[:XROWS, k * D_HID:(k + 1) * D_HID] = jnp.dot(
            x_ref[:, k, :], w_ref[...], preferred_element_type=jnp.float32)
    o_ref[XROWS:, :] = jnp.zeros((PROWS - XROWS, 128), jnp.float32)


_tc_matmul = pl.pallas_call(
    _tc_matmul_body,
    out_shape=jax.ShapeDtypeStruct((PROWS, 128), jnp.float32),
)


def _tc_scale1_body(h_ref, degp_ref, g_ref, norm_ref):
    deg = degp_ref[0] + degp_ref[1]
    norm = lax.rsqrt(jnp.maximum(deg, 1.0))
    norm_ref[...] = norm
    g_ref[...] = h_ref[...] * norm


_tc_scale1 = pl.pallas_call(
    _tc_scale1_body,
    out_shape=[
        jax.ShapeDtypeStruct((PROWS, 128), jnp.float32),
        jax.ShapeDtypeStruct((PROWS, 128), jnp.float32),
    ],
)


def _tc_scale2_body(p_ref, norm_ref, b1_ref, w2bd_ref, g2_ref):
    norm = norm_ref[...]
    h1 = jnp.maximum(norm * (p_ref[0] + p_ref[1]) + b1_ref[...], 0.0)
    g2_ref[...] = jnp.dot(h1, w2bd_ref[...],
                          preferred_element_type=jnp.float32) * norm


_tc_scale2 = pl.pallas_call(
    _tc_scale2_body,
    out_shape=jax.ShapeDtypeStruct((PROWS, 128), jnp.float32),
)


def _tc_scale3_body(p_ref, norm_ref, b2_ref, o_ref):
    o_ref[...] = norm_ref[...] * (p_ref[0] + p_ref[1]) + b2_ref[...]


_tc_scale3 = pl.pallas_call(
    _tc_scale3_body,
    out_shape=jax.ShapeDtypeStruct((PROWS, 128), jnp.float32),
)


def kernel(x, edge_index, W1, b1, W2, b2):
    ei = edge_index.reshape(2, NW, NCHUNK, CHUNK)
    x3 = x.reshape(XROWS, PK, D_FEAT)
    w2p = jnp.zeros((D_HID, D_HID), jnp.float32).at[:, :N_CLASSES].set(W2)
    w2bd = jnp.kron(jnp.eye(PK, dtype=jnp.float32), w2p)
    b1t = jnp.tile(b1, PK).reshape(1, 128)
    b2p = jnp.zeros((D_HID,), jnp.float32).at[:N_CLASSES].set(b2)
    b2t = jnp.tile(b2p, PK).reshape(1, 128)

    h = _tc_matmul(x3, W1)
    degp = _sc_degree(ei)
    g1, norm = _tc_scale1(h, degp)
    p1 = _sc_agg(g1, ei)
    g2 = _tc_scale2(p1, norm, b1t, w2bd)
    p2 = _sc_agg(g2, ei)
    out = _tc_scale3(p2, norm, b2t)
    return out.reshape(NPAD, D_HID)[:N_NODES, :N_CLASSES]


# speedup vs baseline: 1.0238x; 1.0238x over previous
import functools

import jax
import jax.numpy as jnp
from jax import lax
from jax.experimental import pallas as pl
from jax.experimental.pallas import tpu as pltpu
from jax.experimental.pallas import tpu_sc as plsc

N_NODES = 10000
N_EDGES = 320000
D_FEAT = 128
D_HID = 16
N_CLASSES = 7

NC = 2
NS = 16
NW = NC * NS
NPAD = 10240
PK = 8
PROWS = NPAD // PK
XROWS = N_NODES // PK
EPW = N_EDGES // NW
CHUNK = 125
NCHUNK = EPW // CHUNK
NB = 4
ROWS_PT = NPAD // NS
PROWS_PT = PROWS // NS


def _mesh():
    return plsc.VectorSubcoreMesh(core_axis_name="c", subcore_axis_name="s")


_SC_PARAMS = pltpu.CompilerParams(use_tc_tiling_on_sc=False,
                                  needs_layout_passes=False)


@functools.partial(
    pl.kernel,
    out_type=jax.ShapeDtypeStruct((NC, PROWS, 128), jnp.float32),
    mesh=_mesh(),
    compiler_params=_SC_PARAMS,
    scratch_types=[
        pltpu.VMEM((NCHUNK, CHUNK), jnp.int32),
        pltpu.VMEM((CHUNK,), jnp.float32),
        pltpu.VMEM((ROWS_PT,), jnp.float32),
        pltpu.VMEM((PROWS_PT, 128), jnp.float32),
        pltpu.VMEM_SHARED((NPAD,), jnp.float32),
        pltpu.SemaphoreType.DMA,
    ],
)
def _sc_degree(ei_hbm, out_hbm, dstb, onesb, degt, repb, degsh, sem):
    c = lax.axis_index("c")
    s = lax.axis_index("s")
    wid = c * NS + s
    row0 = s * ROWS_PT

    @pl.loop(0, ROWS_PT // 16)
    def _(i):
        degt[pl.ds(i * 16, 16)] = jnp.zeros((16,), jnp.float32)

    for o in (0, 16, 32, 48, 64, 80, 96, CHUNK - 16):
        onesb[pl.ds(o, 16)] = jnp.ones((16,), jnp.float32)

    pltpu.sync_copy(degt, degsh.at[pl.ds(row0, ROWS_PT)])
    pltpu.sync_copy(ei_hbm.at[1, wid], dstb)
    plsc.subcore_barrier()

    @pl.loop(0, NCHUNK, step=8)
    def _(j0):
        @pl.loop(0, 8)
        def _(b):
            pltpu.async_copy(onesb, degsh.at[dstb.at[j0 + b]], sem, add=True)

        @pl.loop(0, 8)
        def _(b):
            pltpu.make_async_copy(onesb, degsh.at[dstb.at[0]], sem).wait()

    plsc.subcore_barrier()

    pltpu.sync_copy(degsh.at[pl.ds(row0, ROWS_PT)], degt)

    @pl.loop(0, ROWS_PT // 16)
    def _(ii):
        d16 = degt[pl.ds(ii * 16, 16)]
        for k in range(16):
            repb[ii * 2 + k // PK, pl.ds((k % PK) * 16, 16)] = (
                jnp.zeros((16,), jnp.float32) + d16[k])

    pltpu.sync_copy(repb, out_hbm.at[c, pl.ds(s * PROWS_PT, PROWS_PT)])


@functools.partial(
    pl.kernel,
    out_type=jax.ShapeDtypeStruct((NC, PROWS, 128), jnp.float32),
    mesh=_mesh(),
    compiler_params=_SC_PARAMS,
    scratch_types=[
        pltpu.VMEM((NCHUNK, CHUNK), jnp.int32),
        pltpu.VMEM((NCHUNK, CHUNK), jnp.int32),
        [pltpu.VMEM((CHUNK, D_HID), jnp.float32) for _ in range(NB)],
        pltpu.VMEM((PROWS_PT, 128), jnp.float32),
        pltpu.VMEM((ROWS_PT, D_HID), jnp.float32),
        pltpu.VMEM_SHARED((NPAD, D_HID), jnp.float32),
        pltpu.VMEM_SHARED((NPAD, D_HID), jnp.float32),
        [pltpu.SemaphoreType.DMA for _ in range(NB)],
        [pltpu.SemaphoreType.DMA for _ in range(NB)],
    ],
)
def _sc_agg(g_hbm, ei_hbm, out_hbm, srcb, dstb, rowsb, pbuf, ubuf,
            aggsh, gsh, gsem, ssem):
    c = lax.axis_index("c")
    s = lax.axis_index("s")
    wid = c * NS + s
    row0 = s * ROWS_PT
    prow0 = s * PROWS_PT

    pltpu.sync_copy(g_hbm.at[pl.ds(prow0, PROWS_PT)], pbuf)
    pltpu.sync_copy(ei_hbm.at[0, wid], srcb)
    pltpu.sync_copy(ei_hbm.at[1, wid], dstb)

    @pl.loop(0, PROWS_PT)
    def _(r):
        for k in range(PK):
            ubuf[r * PK + k, :] = pbuf[r, pl.ds(k * 16, 16)]

    pltpu.sync_copy(ubuf, gsh.at[pl.ds(row0, ROWS_PT)])

    @pl.loop(0, ROWS_PT)
    def _(i):
        ubuf[i, :] = jnp.zeros((D_HID,), jnp.float32)

    pltpu.sync_copy(ubuf, aggsh.at[pl.ds(row0, ROWS_PT)])
    plsc.subcore_barrier()

    def start_gather(j, b):
        pltpu.async_copy(gsh.at[srcb.at[j]], rowsb[b], gsem[b])

    def wait_gather(b):
        pltpu.make_async_copy(gsh.at[srcb.at[0]], rowsb[b], gsem[b]).wait()

    def start_scatter(j, b):
        pltpu.async_copy(rowsb[b], aggsh.at[dstb.at[j]], ssem[b], add=True)

    def wait_scatter(b):
        pltpu.make_async_copy(rowsb[b], aggsh.at[dstb.at[0]], ssem[b]).wait()

    for b in range(NB):
        start_gather(b, b)

    @pl.loop(0, (NCHUNK - NB) // NB)
    def _(gg):
        base = gg * NB
        for b in range(NB):
            j = base + b
            wait_gather(b)
            start_scatter(j, b)
            wait_scatter(b)
            start_gather(j + NB, b)

    for b in range(NB):
        j = NCHUNK - NB + b
        wait_gather(b)
        start_scatter(j, b)
        wait_scatter(b)

    plsc.subcore_barrier()

    pltpu.sync_copy(aggsh.at[pl.ds(row0, ROWS_PT)], ubuf)

    @pl.loop(0, PROWS_PT)
    def _(r):
        for k in range(PK):
            pbuf[r, pl.ds(k * 16, 16)] = ubuf[r * PK + k, :]

    pltpu.sync_copy(pbuf, out_hbm.at[c, pl.ds(prow0, PROWS_PT)])


def _tc_matmul_body(x_ref, w_ref, o_ref):
    for k in range(PK):
        o_ref[:XROWS, k * D_HID:(k + 1) * D_HID] = jnp.dot(
            x_ref[:, k, :], w_ref[...], preferred_element_type=jnp.float32)
    o_ref[XROWS:, :] = jnp.zeros((PROWS - XROWS, 128), jnp.float32)


_tc_matmul = pl.pallas_call(
    _tc_matmul_body,
    out_shape=jax.ShapeDtypeStruct((PROWS, 128), jnp.float32),
)


def _tc_scale1_body(h_ref, degp_ref, g_ref, norm_ref):
    deg = degp_ref[0] + degp_ref[1]
    norm = lax.rsqrt(jnp.maximum(deg, 1.0))
    norm_ref[...] = norm
    g_ref[...] = h_ref[...] * norm


_tc_scale1 = pl.pallas_call(
    _tc_scale1_body,
    out_shape=[
        jax.ShapeDtypeStruct((PROWS, 128), jnp.float32),
        jax.ShapeDtypeStruct((PROWS, 128), jnp.float32),
    ],
)


def _tc_scale2_body(p_ref, norm_ref, b1_ref, w2bd_ref, g2_ref):
    norm = norm_ref[...]
    h1 = jnp.maximum(norm * (p_ref[0] + p_ref[1]) + b1_ref[...], 0.0)
    g2_ref[...] = jnp.dot(h1, w2bd_ref[...],
                          preferred_element_type=jnp.float32) * norm


_tc_scale2 = pl.pallas_call(
    _tc_scale2_body,
    out_shape=jax.ShapeDtypeStruct((PROWS, 128), jnp.float32),
)


def _tc_scale3_body(p_ref, norm_ref, b2_ref, o_ref):
    o_ref[...] = norm_ref[...] * (p_ref[0] + p_ref[1]) + b2_ref[...]


_tc_scale3 = pl.pallas_call(
    _tc_scale3_body,
    out_shape=jax.ShapeDtypeStruct((PROWS, 128), jnp.float32),
)


def kernel(x, edge_index, W1, b1, W2, b2):
    ei = edge_index.reshape(2, NW, NCHUNK, CHUNK)
    x3 = x.reshape(XROWS, PK, D_FEAT)
    w2p = jnp.zeros((D_HID, D_HID), jnp.float32).at[:, :N_CLASSES].set(W2)
    w2bd = jnp.kron(jnp.eye(PK, dtype=jnp.float32), w2p)
    b1t = jnp.tile(b1, PK).reshape(1, 128)
    b2p = jnp.zeros((D_HID,), jnp.float32).at[:N_CLASSES].set(b2)
    b2t = jnp.tile(b2p, PK).reshape(1, 128)

    h = _tc_matmul(x3, W1)
    degp = _sc_degree(ei)
    g1, norm = _tc_scale1(h, degp)
    p1 = _sc_agg(g1, ei)
    g2 = _tc_scale2(p1, norm, b1t, w2bd)
    p2 = _sc_agg(g2, ei)
    out = _tc_scale3(p2, norm, b2t)
    return out.reshape(NPAD, D_HID)[:N_NODES, :N_CLASSES]
